# baseline (speedup 1.0000x reference)
import jax
import jax.numpy as jnp
from jax import lax
from jax.experimental import pallas as pl
from jax.experimental.pallas import tpu as pltpu
from jax.experimental.pallas import tpu_sc as plsc

B = 4
S = 8192
D = 1024
L = 16
NC = 2
NS = 16
NW = NC * NS
R = B * S
RPW = R // NW
CH = 16
NCH = RPW // CH
NBUF = 4
AHEAD = 3


def _body(xf, sgn, perm, out, perm_v, sgn_v, s_v, bufs, gsems, ssems):
    wid = lax.axis_index("s") * NC + lax.axis_index("c")
    wbase = wid * RPW
    bofs = (wbase // S) * S
    ibase = wbase - bofs

    pltpu.sync_copy(perm.at[pl.ds(ibase, RPW)], perm_v)
    pltpu.sync_copy(sgn, sgn_v)

    vofs = jnp.full((L,), bofs, jnp.int32)

    def prep(k, _):
        idx16 = perm_v[pl.ds(k * L, L)]
        s_v[pl.ds(k * L, L)] = plsc.load_gather(sgn_v, [idx16])
        perm_v[pl.ds(k * L, L)] = idx16 + vofs
        return 0

    lax.fori_loop(0, RPW // L, prep, 0, unroll=2)

    def start_gather(c, b):
        pltpu.async_copy(xf.at[perm_v.at[pl.ds(c * CH, CH)]], bufs[b], gsems[b])

    def wait_sem(sem, b):
        pltpu.make_async_copy(xf.at[pl.ds(0, CH)], bufs[b], sem).wait()

    for c in range(AHEAD):
        start_gather(c, c)

    def group(g, _):
        c0 = g * NBUF
        for b in range(NBUF):
            c = c0 + b
            bn = (b + AHEAD) % NBUF

            @pl.when(c >= NBUF - AHEAD)
            def _():
                wait_sem(ssems[bn], bn)

            @pl.when(c + AHEAD < NCH)
            def _():
                start_gather(c + AHEAD, bn)

            wait_sem(gsems[b], b)

            def row(r, _):
                svec = plsc.load_gather(
                    s_v, [jnp.full((L,), c * CH + r, jnp.int32)])
                for j in range(D // L):
                    sl = pl.ds(j * L, L)
                    bufs[b][r, sl] = bufs[b][r, sl] * svec
                return 0

            lax.fori_loop(0, CH, row, 0)
            pltpu.async_copy(bufs[b], out.at[pl.ds(wbase + c * CH, CH)],
                             ssems[b])
        return 0

    lax.fori_loop(0, NCH // NBUF, group, 0)

    for c in range(NCH - (NBUF - AHEAD), NCH):
        wait_sem(ssems[c % NBUF], c % NBUF)


@jax.jit
def kernel(x, signs, perm):
    xf = x.reshape(R, D)
    sgn = signs.reshape(S)
    mesh = plsc.VectorSubcoreMesh(core_axis_name="c", subcore_axis_name="s",
                                  num_cores=NC, num_subcores=NS)
    out = pl.kernel(
        _body,
        out_type=jax.ShapeDtypeStruct((R, D), jnp.float32),
        mesh=mesh,
        scratch_types=[
            pltpu.VMEM((RPW,), jnp.int32),
            pltpu.VMEM((S,), jnp.float32),
            pltpu.VMEM((RPW,), jnp.float32),
            [pltpu.VMEM((CH, D), jnp.float32) for _ in range(NBUF)],
            [pltpu.SemaphoreType.DMA for _ in range(NBUF)],
            [pltpu.SemaphoreType.DMA for _ in range(NBUF)],
        ],
        compiler_params=pltpu.CompilerParams(needs_layout_passes=False),
    )(xf, sgn, perm)
    return out.reshape(B, S, D)


# speedup vs baseline: 1.2350x; 1.2350x over previous
import jax
import jax.numpy as jnp
from jax import lax
from jax.experimental import pallas as pl
from jax.experimental.pallas import tpu as pltpu
from jax.experimental.pallas import tpu_sc as plsc

B = 4
S = 8192
D = 1024
L = 16
NC = 2
NS = 16
NW = NC * NS
R = B * S
RPW = R // NW
CH = 8
NCH = RPW // CH
NBUF = 8
AHEAD = 4


def _body(xf, sgn, perm, out, perm_v, sgn_v, s_v, bufs, gsems, ssems):
    wid = lax.axis_index("s") * NC + lax.axis_index("c")
    wbase = wid * RPW
    bofs = (wbase // S) * S
    ibase = wbase - bofs

    pltpu.sync_copy(perm.at[pl.ds(ibase, RPW)], perm_v)
    pltpu.sync_copy(sgn, sgn_v)

    vofs = jnp.full((L,), bofs, jnp.int32)

    def prep(k, _):
        idx16 = perm_v[pl.ds(k * L, L)]
        s_v[pl.ds(k * L, L)] = plsc.load_gather(sgn_v, [idx16])
        perm_v[pl.ds(k * L, L)] = idx16 + vofs
        return 0

    lax.fori_loop(0, RPW // L, prep, 0, unroll=2)

    def start_gather(c, b):
        pltpu.async_copy(xf.at[perm_v.at[pl.ds(c * CH, CH)]], bufs[b], gsems[b])

    def wait_sem(sem, b):
        pltpu.make_async_copy(xf.at[pl.ds(0, CH)], bufs[b], sem).wait()

    for c in range(AHEAD):
        start_gather(c, c)

    def group(g, _):
        c0 = g * NBUF
        for b in range(NBUF):
            c = c0 + b
            bn = (b + AHEAD) % NBUF

            @pl.when(c >= NBUF - AHEAD)
            def _():
                wait_sem(ssems[bn], bn)

            @pl.when(c + AHEAD < NCH)
            def _():
                start_gather(c + AHEAD, bn)

            wait_sem(gsems[b], b)

            def row(r, _):
                svec = plsc.load_gather(
                    s_v, [jnp.full((L,), c * CH + r, jnp.int32)])
                for j in range(D // L):
                    sl = pl.ds(j * L, L)
                    bufs[b][r, sl] = bufs[b][r, sl] * svec
                return 0

            lax.fori_loop(0, CH, row, 0)
            pltpu.async_copy(bufs[b], out.at[pl.ds(wbase + c * CH, CH)],
                             ssems[b])
        return 0

    lax.fori_loop(0, NCH // NBUF, group, 0)

    for c in range(NCH - (NBUF - AHEAD), NCH):
        wait_sem(ssems[c % NBUF], c % NBUF)


@jax.jit
def kernel(x, signs, perm):
    xf = x.reshape(R, D)
    sgn = signs.reshape(S)
    mesh = plsc.VectorSubcoreMesh(core_axis_name="c", subcore_axis_name="s",
                                  num_cores=NC, num_subcores=NS)
    out = pl.kernel(
        _body,
        out_type=jax.ShapeDtypeStruct((R, D), jnp.float32),
        mesh=mesh,
        scratch_types=[
            pltpu.VMEM((RPW,), jnp.int32),
            pltpu.VMEM((S,), jnp.float32),
            pltpu.VMEM((RPW,), jnp.float32),
            [pltpu.VMEM((CH, D), jnp.float32) for _ in range(NBUF)],
            [pltpu.SemaphoreType.DMA for _ in range(NBUF)],
            [pltpu.SemaphoreType.DMA for _ in range(NBUF)],
        ],
        compiler_params=pltpu.CompilerParams(needs_layout_passes=False),
    )(xf, sgn, perm)
    return out.reshape(B, S, D)


# speedup vs baseline: 1.2353x; 1.0002x over previous
import jax
import jax.numpy as jnp
from jax import lax
from jax.experimental import pallas as pl
from jax.experimental.pallas import tpu as pltpu
from jax.experimental.pallas import tpu_sc as plsc

B = 4
S = 8192
D = 1024
L = 16
NC = 2
NS = 16
NW = NC * NS
R = B * S
RPW = R // NW
CH = 16
NCH = RPW // CH
NBUF = 4
AHEAD = 2


def _body(xf, sgn, perm, out, perm_v, sgn_v, s_v, bufs, gsems, ssems):
    wid = lax.axis_index("s") * NC + lax.axis_index("c")
    wbase = wid * RPW
    bofs = (wbase // S) * S
    ibase = wbase - bofs

    pltpu.sync_copy(perm.at[pl.ds(ibase, RPW)], perm_v)
    pltpu.sync_copy(sgn, sgn_v)

    vofs = jnp.full((L,), bofs, jnp.int32)

    def prep(k, _):
        idx16 = perm_v[pl.ds(k * L, L)]
        s_v[pl.ds(k * L, L)] = plsc.load_gather(sgn_v, [idx16])
        perm_v[pl.ds(k * L, L)] = idx16 + vofs
        return 0

    lax.fori_loop(0, RPW // L, prep, 0, unroll=2)

    def start_gather(c, b):
        pltpu.async_copy(xf.at[perm_v.at[pl.ds(c * CH, CH)]], bufs[b], gsems[b])

    def wait_sem(sem, b):
        pltpu.make_async_copy(xf.at[pl.ds(0, CH)], bufs[b], sem).wait()

    for c in range(AHEAD):
        start_gather(c, c)

    def group(g, _):
        c0 = g * NBUF
        for b in range(NBUF):
            c = c0 + b
            bn = (b + AHEAD) % NBUF

            @pl.when(c >= NBUF - AHEAD)
            def _():
                wait_sem(ssems[bn], bn)

            @pl.when(c + AHEAD < NCH)
            def _():
                start_gather(c + AHEAD, bn)

            wait_sem(gsems[b], b)

            def row(r, _):
                svec = plsc.load_gather(
                    s_v, [jnp.full((L,), c * CH + r, jnp.int32)])
                for j in range(D // L):
                    sl = pl.ds(j * L, L)
                    bufs[b][r, sl] = bufs[b][r, sl] * svec
                return 0

            lax.fori_loop(0, CH, row, 0)
            pltpu.async_copy(bufs[b], out.at[pl.ds(wbase + c * CH, CH)],
                             ssems[b])
        return 0

    lax.fori_loop(0, NCH // NBUF, group, 0)

    for c in range(NCH - (NBUF - AHEAD), NCH):
        wait_sem(ssems[c % NBUF], c % NBUF)


@jax.jit
def kernel(x, signs, perm):
    xf = x.reshape(R, D)
    sgn = signs.reshape(S)
    mesh = plsc.VectorSubcoreMesh(core_axis_name="c", subcore_axis_name="s",
                                  num_cores=NC, num_subcores=NS)
    out = pl.kernel(
        _body,
        out_type=jax.ShapeDtypeStruct((R, D), jnp.float32),
        mesh=mesh,
        scratch_types=[
            pltpu.VMEM((RPW,), jnp.int32),
            pltpu.VMEM((S,), jnp.float32),
            pltpu.VMEM((RPW,), jnp.float32),
            [pltpu.VMEM((CH, D), jnp.float32) for _ in range(NBUF)],
            [pltpu.SemaphoreType.DMA for _ in range(NBUF)],
            [pltpu.SemaphoreType.DMA for _ in range(NBUF)],
        ],
        compiler_params=pltpu.CompilerParams(needs_layout_passes=False),
    )(xf, sgn, perm)
    return out.reshape(B, S, D)
